# Initial kernel scaffold; baseline (speedup 1.0000x reference)
import jax
import jax.numpy as jnp
from jax import lax
from jax.experimental import pallas as pl
from jax.experimental.pallas import tpu as pltpu

_V = 100000
_D = 300
_G = 100
_B = 100
_K = _G + _B
_BV = 2000
_NB = _V // _BV

_SHIFT = 1 << 20
_MASK = _SHIFT - 1


def _dist_body(w_ref, x_ref, idx_out, clue_out, counts_out, best_ref):
    i = pl.program_id(0)
    x = x_ref[...]
    w = w_ref[...]
    p = lax.dot_general(x, w, (((1,), (1,)), ((), ())),
                        preferred_element_type=jnp.float32)
    c2 = jnp.sum(w * w, axis=1)
    t = c2[None, :] - 2.0 * p
    m = jnp.min(t[:, _G:], axis=1)
    good_close = (t[:, :_G] < m[:, None]).astype(jnp.int32)
    counts = jnp.sum(good_close, axis=1)
    counts_out[...] = counts

    rows = i * _BV + lax.broadcasted_iota(jnp.int32, (_BV, 1), 0)
    combined = counts[:, None] * _SHIFT + (_MASK - rows)
    bmax = jnp.max(combined)

    @pl.when(i == 0)
    def _():
        best_ref[0] = bmax

    @pl.when(i > 0)
    def _():
        best_ref[0] = jnp.maximum(best_ref[0], bmax)

    @pl.when(i == _NB - 1)
    def _():
        best = best_ref[0]
        clue_out[0, 0] = best // _SHIFT
        idx_out[0, 0] = _MASK - (best & _MASK)


def _distance_pass(w, embeddings, interpret=False):
    return pl.pallas_call(
        _dist_body,
        grid=(_NB,),
        in_specs=[
            pl.BlockSpec((_K, _D), lambda i: (0, 0)),
            pl.BlockSpec((_BV, _D), lambda i: (i, 0)),
        ],
        out_specs=[
            pl.BlockSpec(memory_space=pltpu.SMEM),
            pl.BlockSpec(memory_space=pltpu.SMEM),
            pl.BlockSpec((_BV,), lambda i: (i,)),
        ],
        out_shape=[
            jax.ShapeDtypeStruct((1, 1), jnp.int32),
            jax.ShapeDtypeStruct((1, 1), jnp.int32),
            jax.ShapeDtypeStruct((_V,), jnp.int32),
        ],
        scratch_shapes=[pltpu.SMEM((1,), jnp.int32)],
        interpret=interpret,
    )(w, embeddings)


def kernel(embeddings, good_idx, bad_idx):
    cat_idx = jnp.concatenate([good_idx, bad_idx]).astype(jnp.int32)
    w = jnp.take(embeddings, cat_idx, axis=0)
    idx, clue, counts = _distance_pass(w, embeddings)
    return (idx[0, 0], clue[0, 0], counts)


# speedup vs baseline: 2.3533x; 2.3533x over previous
import jax
import jax.numpy as jnp
from jax import lax
from jax.experimental import pallas as pl
from jax.experimental.pallas import tpu as pltpu

_V = 100000
_D = 300
_G = 100
_B = 100
_K = _G + _B
_BV = 2000
_NB = _V // _BV

_SHIFT = 1 << 20
_MASK = _SHIFT - 1


def _dist_body(wgt_ref, wbt_ref, x_ref, idx_out, clue_out, counts_out, best_ref):
    i = pl.program_id(0)
    x = x_ref[...]
    wgt = wgt_ref[...]
    wbt = wbt_ref[...]
    pg = jnp.dot(x, wgt, preferred_element_type=jnp.float32)
    pb = jnp.dot(x, wbt, preferred_element_type=jnp.float32)
    g2 = jnp.sum(wgt * wgt, axis=0, keepdims=True)
    b2 = jnp.sum(wbt * wbt, axis=0, keepdims=True)
    tg = g2 - 2.0 * pg
    tb = b2 - 2.0 * pb
    m = jnp.min(tb, axis=1, keepdims=True)
    good_close = (tg < m).astype(jnp.int32)
    counts = jnp.sum(good_close, axis=1, keepdims=True)
    counts_out[...] = counts[None]

    rows = i * _BV + lax.broadcasted_iota(jnp.int32, (_BV, 1), 0)
    combined = counts * _SHIFT + (_MASK - rows)
    bmax = jnp.max(combined)

    @pl.when(i == 0)
    def _():
        best_ref[0] = bmax

    @pl.when(i > 0)
    def _():
        best_ref[0] = jnp.maximum(best_ref[0], bmax)

    @pl.when(i == _NB - 1)
    def _():
        best = best_ref[0]
        clue_out[0, 0] = best // _SHIFT
        idx_out[0, 0] = _MASK - (best & _MASK)


def _distance_pass(wg, wb, embeddings, interpret=False):
    return pl.pallas_call(
        _dist_body,
        grid=(_NB,),
        in_specs=[
            pl.BlockSpec((_D, _G), lambda i: (0, 0)),
            pl.BlockSpec((_D, _B), lambda i: (0, 0)),
            pl.BlockSpec((_BV, _D), lambda i: (i, 0)),
        ],
        out_specs=[
            pl.BlockSpec(memory_space=pltpu.SMEM),
            pl.BlockSpec(memory_space=pltpu.SMEM),
            pl.BlockSpec((1, _BV, 1), lambda i: (i, 0, 0)),
        ],
        out_shape=[
            jax.ShapeDtypeStruct((1, 1), jnp.int32),
            jax.ShapeDtypeStruct((1, 1), jnp.int32),
            jax.ShapeDtypeStruct((_NB, _BV, 1), jnp.int32),
        ],
        scratch_shapes=[pltpu.SMEM((1,), jnp.int32)],
        interpret=interpret,
    )(wg, wb, embeddings)


def kernel(embeddings, good_idx, bad_idx):
    wgt = jnp.take(embeddings, good_idx.astype(jnp.int32), axis=0).T
    wbt = jnp.take(embeddings, bad_idx.astype(jnp.int32), axis=0).T
    idx, clue, counts = _distance_pass(wgt, wbt, embeddings)
    return (idx[0, 0], clue[0, 0], counts.reshape(_V))


# speedup vs baseline: 2.5376x; 1.0783x over previous
import jax
import jax.numpy as jnp
from jax import lax
from jax.experimental import pallas as pl
from jax.experimental.pallas import tpu as pltpu

_V = 100000
_D = 300
_G = 100
_B = 100
_K = _G + _B
_BV = 5000
_NB = _V // _BV

_SHIFT = 1 << 20
_MASK = _SHIFT - 1


def _dist_body(wgt_ref, wbt_ref, x_ref, idx_out, clue_out, counts_out, best_ref):
    i = pl.program_id(0)
    x = x_ref[...]
    wgt = wgt_ref[...]
    wbt = wbt_ref[...]
    pg = jnp.dot(x, wgt, preferred_element_type=jnp.float32)
    pb = jnp.dot(x, wbt, preferred_element_type=jnp.float32)
    g2 = jnp.sum(wgt * wgt, axis=0, keepdims=True)
    b2 = jnp.sum(wbt * wbt, axis=0, keepdims=True)
    tg = g2 - 2.0 * pg
    tb = b2 - 2.0 * pb
    m = jnp.min(tb, axis=1, keepdims=True)
    good_close = (tg < m).astype(jnp.int32)
    counts = jnp.sum(good_close, axis=1, keepdims=True)
    counts_out[...] = counts[None]

    rows = i * _BV + lax.broadcasted_iota(jnp.int32, (_BV, 1), 0)
    combined = counts * _SHIFT + (_MASK - rows)
    bmax = jnp.max(combined)

    @pl.when(i == 0)
    def _():
        best_ref[0] = bmax

    @pl.when(i > 0)
    def _():
        best_ref[0] = jnp.maximum(best_ref[0], bmax)

    @pl.when(i == _NB - 1)
    def _():
        best = best_ref[0]
        clue_out[0, 0] = best // _SHIFT
        idx_out[0, 0] = _MASK - (best & _MASK)


def _distance_pass(wg, wb, embeddings, interpret=False):
    return pl.pallas_call(
        _dist_body,
        grid=(_NB,),
        in_specs=[
            pl.BlockSpec((_D, _G), lambda i: (0, 0)),
            pl.BlockSpec((_D, _B), lambda i: (0, 0)),
            pl.BlockSpec((_BV, _D), lambda i: (i, 0)),
        ],
        out_specs=[
            pl.BlockSpec(memory_space=pltpu.SMEM),
            pl.BlockSpec(memory_space=pltpu.SMEM),
            pl.BlockSpec((1, _BV, 1), lambda i: (i, 0, 0)),
        ],
        out_shape=[
            jax.ShapeDtypeStruct((1, 1), jnp.int32),
            jax.ShapeDtypeStruct((1, 1), jnp.int32),
            jax.ShapeDtypeStruct((_NB, _BV, 1), jnp.int32),
        ],
        scratch_shapes=[pltpu.SMEM((1,), jnp.int32)],
        interpret=interpret,
    )(wg, wb, embeddings)


def kernel(embeddings, good_idx, bad_idx):
    wgt = jnp.take(embeddings, good_idx.astype(jnp.int32), axis=0).T
    wbt = jnp.take(embeddings, bad_idx.astype(jnp.int32), axis=0).T
    idx, clue, counts = _distance_pass(wgt, wbt, embeddings)
    return (idx[0, 0], clue[0, 0], counts.reshape(_V))
